# baseline (speedup 1.0000x reference)
import functools

import jax
import jax.numpy as jnp
from jax import lax
from jax.experimental import pallas as pl
from jax.experimental.pallas import tpu as pltpu
from jax.experimental.pallas import tpu_sc as plsc

B = 16384
NF = 32
NBIN = 30
L = 16
NC = 2
NS = 16
NW = NC * NS
BPW = B // NW
NCH = BPW // L

_LN2 = 0.6931471805599453
_SQRT2 = 1.4142135623730951
_BETA = 0.4


def _body(user_r, item_r, tbin_r, tday_r, mc_r, mean_r, bu_r, alpha_r,
          aukT_r, bcu_r, wpuT_r, wpiT_r, bi_r, wbitT_r, pkutT_r, btday_r,
          wcu_r, gm_r, out_r,
          u_v, i_v, tb_v, td_v, mc_v,
          bu_v, al_v, me_v, bc_v, bi_v, btd_v, wcu_v,
          wpu_v, auk_v, wpi_v, pkut_v, wbit_v,
          gm_v, out_v, sem):
    wid = lax.axis_index("s") * NC + lax.axis_index("c")
    base = wid * BPW

    pltpu.sync_copy(user_r.at[pl.ds(base, BPW)], u_v)
    pltpu.sync_copy(item_r.at[pl.ds(base, BPW)], i_v)
    pltpu.sync_copy(tbin_r.at[pl.ds(base, BPW)], tb_v)
    pltpu.sync_copy(tday_r.at[pl.ds(base, BPW)], td_v)
    pltpu.sync_copy(mc_r.at[pl.ds(base, BPW)], mc_v)
    pltpu.sync_copy(gm_r, gm_v)

    copies = []
    copies.append(pltpu.async_copy(bu_r.at[u_v], bu_v, sem))
    copies.append(pltpu.async_copy(alpha_r.at[u_v], al_v, sem))
    copies.append(pltpu.async_copy(mean_r.at[u_v], me_v, sem))
    copies.append(pltpu.async_copy(bcu_r.at[u_v], bc_v, sem))
    copies.append(pltpu.async_copy(bi_r.at[i_v], bi_v, sem))
    copies.append(pltpu.async_copy(btday_r.at[mc_v], btd_v, sem))
    copies.append(pltpu.async_copy(wcu_r.at[mc_v], wcu_v, sem))
    for f in range(NF):
        copies.append(pltpu.async_copy(wpuT_r.at[f].at[u_v], wpu_v.at[f],
                                       sem))
        copies.append(pltpu.async_copy(aukT_r.at[f].at[u_v], auk_v.at[f],
                                       sem))
        copies.append(pltpu.async_copy(wpiT_r.at[f].at[i_v], wpi_v.at[f],
                                       sem))
        copies.append(pltpu.async_copy(pkutT_r.at[f].at[mc_v], pkut_v.at[f],
                                       sem))
    for t in range(NBIN):
        copies.append(pltpu.async_copy(wbitT_r.at[t].at[i_v], wbit_v.at[t],
                                       sem))
    for c in copies:
        c.wait()

    gm16 = gm_v[...]

    def chunk_body(k, c):
        b16 = k * L
        sl = pl.ds(b16, L)
        diff = td_v[sl].astype(jnp.float32) - me_v[sl]
        sgn = jnp.sign(diff)
        t = jnp.abs(diff)
        bits = lax.bitcast_convert_type(t, jnp.int32)
        e_i = (bits >> 23) - 127
        m = lax.bitcast_convert_type((bits & 0x7FFFFF) | 0x3F800000,
                                     jnp.float32)
        big = m > _SQRT2
        m = jnp.where(big, m * 0.5, m)
        e_f = e_i.astype(jnp.float32) + jnp.where(big, 1.0, 0.0)
        z = (m - 1.0) / (m + 1.0)
        z2 = z * z
        poly = 1.0 + z2 * ((1.0 / 3.0) + z2 * ((1.0 / 5.0) + z2 * (1.0 / 7.0)))
        ln_t = e_f * _LN2 + 2.0 * z * poly
        devt = sgn * jnp.exp(_BETA * ln_t)

        e16 = b16 + lax.iota(jnp.int32, L)
        wb16 = plsc.load_gather(wbit_v, [tb_v[sl], e16])
        acc = (gm16 + bu_v[sl] + al_v[sl] * devt + btd_v[sl]
               + (bi_v[sl] + wb16) * (bc_v[sl] + wcu_v[sl]))
        for f in range(NF):
            acc = acc + (wpu_v[f, sl] + auk_v[f, sl] * devt
                         + pkut_v[f, sl]) * wpi_v[f, sl]
        out_v[sl] = acc
        return c

    lax.fori_loop(0, NCH, chunk_body, 0)

    pltpu.sync_copy(out_v, out_r.at[pl.ds(base, BPW)])


@jax.jit
def _run(user, item, tbin, tday, mc, mean_ud, bu, alpha, aukT, bcu,
         wpuT, wpiT, bi, wbitT, pkutT, btday, wcu, gm16):
    mesh = plsc.VectorSubcoreMesh(core_axis_name="c", subcore_axis_name="s")
    f = functools.partial(
        pl.kernel,
        out_type=jax.ShapeDtypeStruct((B,), jnp.float32),
        mesh=mesh,
        compiler_params=pltpu.CompilerParams(needs_layout_passes=False,
                                             use_tc_tiling_on_sc=False),
        scratch_types=[
            pltpu.VMEM((BPW,), jnp.int32),
            pltpu.VMEM((BPW,), jnp.int32),
            pltpu.VMEM((BPW,), jnp.int32),
            pltpu.VMEM((BPW,), jnp.int32),
            pltpu.VMEM((BPW,), jnp.int32),
            pltpu.VMEM((BPW,), jnp.float32),
            pltpu.VMEM((BPW,), jnp.float32),
            pltpu.VMEM((BPW,), jnp.float32),
            pltpu.VMEM((BPW,), jnp.float32),
            pltpu.VMEM((BPW,), jnp.float32),
            pltpu.VMEM((BPW,), jnp.float32),
            pltpu.VMEM((BPW,), jnp.float32),
            pltpu.VMEM((NF, BPW), jnp.float32),
            pltpu.VMEM((NF, BPW), jnp.float32),
            pltpu.VMEM((NF, BPW), jnp.float32),
            pltpu.VMEM((NF, BPW), jnp.float32),
            pltpu.VMEM((NBIN, BPW), jnp.float32),
            pltpu.VMEM((L,), jnp.float32),
            pltpu.VMEM((BPW,), jnp.float32),
            pltpu.SemaphoreType.DMA,
        ],
    )(_body)
    return f(user, item, tbin, tday, mc, mean_ud, bu, alpha, aukT, bcu,
             wpuT, wpiT, bi, wbitT, pkutT, btday, wcu, gm16)


def kernel(user, item, tbin, tday, mean_ud, global_mean, maxday_cat,
           WPI, WPU, BU, BI, WBIT, Alpha, AlphaUK, WPUKT, BTDay, BCU, WCU):
    gm16 = jnp.broadcast_to(jnp.float32(global_mean), (L,))
    return _run(user.astype(jnp.int32), item.astype(jnp.int32),
                tbin.astype(jnp.int32), tday.astype(jnp.int32),
                maxday_cat.astype(jnp.int32), mean_ud, BU, Alpha,
                jnp.transpose(AlphaUK), BCU, jnp.transpose(WPU),
                jnp.transpose(WPI), BI, jnp.transpose(WBIT),
                jnp.transpose(WPUKT), BTDay, WCU, gm16)
